# Initial kernel scaffold; baseline (speedup 1.0000x reference)
import jax
import jax.numpy as jnp
from jax.experimental import pallas as pl


def kernel(abnr_feat_magn, norm_feat_magn, abnr_feats, norm_feats, abnr_sls, norm_sls, ldata):
    raise NotImplementedError("write your pallas kernel here")


# speedup vs baseline: 1.6965x; 1.6965x over previous
import functools

import jax
import jax.numpy as jnp
from jax import lax
from jax.experimental import pallas as pl
from jax.experimental.pallas import tpu as pltpu
from jax.experimental.pallas import tpu_sc as plsc

_ALPHA = 0.0001
_MARGIN = 100.0
_K = 3
_NC, _B, _T, _F = 2, 16, 4096, 256
_L = 16
_NEG = -3.0e38
_BIGI = 1 << 30


def _sc_body(magn_hbm, sls_hbm, afl_hbm, nfl_hbm, out_hbm,
             magn_v, sls_v, idx_v, rows_a, rows_n, out_v, sem_a, sem_n):
    cid = lax.axis_index("c")
    sid = lax.axis_index("s")
    wid = sid * 2 + cid
    t = wid // _B
    b = wid % _B
    lane = jnp.arange(_L, dtype=jnp.int32)

    pltpu.sync_copy(magn_hbm.at[t, b], magn_v)
    pltpu.sync_copy(sls_hbm.at[t, b], sls_v)

    def _argmax_pop():
        def body(i, carry):
            mv, mi = carry
            v = magn_v[pl.ds(i * _L, _L)]
            gi = i * _L + lane
            better = v > mv
            return jnp.where(better, v, mv), jnp.where(better, gi, mi)
        mv, mi = lax.fori_loop(
            0, _T // _L, body,
            (jnp.full((_L,), _NEG, jnp.float32), jnp.zeros((_L,), jnp.int32)))
        m = jnp.max(mv)
        gidx = jnp.min(jnp.where(mv == m, mi, _BIGI))
        plsc.store_scatter(magn_v, [jnp.full((_L,), gidx, jnp.int32)],
                           jnp.full((_L,), _NEG, jnp.float32),
                           mask=lane == 0)
        return gidx

    i0 = _argmax_pop()
    i1 = _argmax_pop()
    i2 = _argmax_pop()

    sls_sel = plsc.load_gather(
        sls_v, [jnp.where(lane == 0, i0, jnp.where(lane == 1, i1, i2))])
    vls = jnp.sum(jnp.where(lane < _K, sls_sel, 0.0)) * jnp.float32(1.0 / _K)

    r0 = b * _T
    r1 = (_B + b) * _T
    fidx = jnp.where(lane == 0, r0 + i0,
           jnp.where(lane == 1, r0 + i1,
           jnp.where(lane == 2, r0 + i2,
           jnp.where(lane == 3, r1 + i0,
           jnp.where(lane == 4, r1 + i1,
           jnp.where(lane == 5, r1 + i2, r0 + i0))))))
    idx_v[...] = fidx

    cp_a = pltpu.async_copy(afl_hbm.at[idx_v], rows_a, sem_a)
    cp_n = pltpu.async_copy(nfl_hbm.at[idx_v], rows_n, sem_n)
    cp_a.wait()
    cp_n.wait()

    tmask = jnp.full((_L,), 0, jnp.int32) + t == 0

    def pick(r, s):
        return jnp.where(tmask, rows_a[r, s], rows_n[r, s])

    third = jnp.float32(1.0 / 3.0)
    acc0 = jnp.zeros((_L,), jnp.float32)
    acc1 = jnp.zeros((_L,), jnp.float32)
    for c in range(_F // _L):
        s = pl.ds(c * _L, _L)
        m0 = (pick(0, s) + pick(1, s) + pick(2, s)) * third
        acc0 = acc0 + m0 * m0
        m1 = (pick(3, s) + pick(4, s) + pick(5, s)) * third
        acc1 = acc1 + m1 * m1
    ssq0 = jnp.sum(acc0)
    ssq1 = jnp.sum(acc1)

    out_v[...] = jnp.where(lane == 0, ssq0,
                 jnp.where(lane == 1, ssq1,
                 jnp.where(lane == 2, vls, 0.0)))
    pltpu.sync_copy(out_v, out_hbm.at[t, b])


_sc_call = functools.partial(
    pl.kernel,
    mesh=plsc.VectorSubcoreMesh(core_axis_name="c", subcore_axis_name="s",
                                num_cores=2, num_subcores=16),
    out_type=jax.ShapeDtypeStruct((_NC, _B, _L), jnp.float32),
    scratch_types=[
        pltpu.VMEM((_T,), jnp.float32),
        pltpu.VMEM((_T,), jnp.float32),
        pltpu.VMEM((_L,), jnp.int32),
        pltpu.VMEM((_L, _F), jnp.float32),
        pltpu.VMEM((_L, _F), jnp.float32),
        pltpu.VMEM((_L,), jnp.float32),
        pltpu.SemaphoreType.DMA,
        pltpu.SemaphoreType.DMA,
    ],
    compiler_params=pltpu.CompilerParams(needs_layout_passes=False),
)(_sc_body)


def _tc_body(x_ref, o_ref):
    x = x_ref[...]
    la = jnp.abs(_MARGIN - jnp.sqrt(x[0, :, 0:2]))
    ln = jnp.sqrt(x[1, :, 0:2])
    loss_rtfm = jnp.mean((la + ln) ** 2)
    vls_abn = x[0, :, 2]
    vls_norm = x[1, :, 2]
    bcea = -jnp.mean(jnp.maximum(jnp.log(vls_abn), -100.0))
    bcen = -jnp.mean(jnp.maximum(jnp.log(1.0 - vls_norm), -100.0))
    o_ref[0] = _ALPHA * loss_rtfm
    o_ref[1] = bcea + bcen


_tc_call = pl.pallas_call(
    _tc_body,
    out_shape=jax.ShapeDtypeStruct((2,), jnp.float32),
    out_specs=pl.BlockSpec(memory_space=pltpu.SMEM),
)


def kernel(abnr_feat_magn, norm_feat_magn, abnr_feats, norm_feats,
           abnr_sls, norm_sls, ldata):
    magn = jnp.stack([abnr_feat_magn, norm_feat_magn])
    sls = jnp.stack([abnr_sls, norm_sls])
    afl = abnr_feats.reshape(_NC * _B * _T, _F)
    nfl = norm_feats.reshape(_NC * _B * _T, _F)
    part = _sc_call(magn, sls, afl, nfl)
    return _tc_call(part)


# speedup vs baseline: 1.8777x; 1.1068x over previous
import functools

import jax
import jax.numpy as jnp
from jax import lax
from jax.experimental import pallas as pl
from jax.experimental.pallas import tpu as pltpu
from jax.experimental.pallas import tpu_sc as plsc

_ALPHA = 0.0001
_MARGIN = 100.0
_K = 3
_NC, _B, _T, _F = 2, 16, 4096, 256
_L = 16
_NEG = -3.0e38
_BIGI = 1 << 30
_UNROLL = 8


def _sc_body(magn_hbm, sls_hbm, afl_hbm, nfl_hbm,
             out_hbm, magn_v, sls_v, idx_v, rows_a, rows_n, out_v,
             sem_m, sem_s, sem_a, sem_n):
    cid = lax.axis_index("c")
    sid = lax.axis_index("s")
    wid = sid * 2 + cid
    t = wid // _B
    b = wid % _B
    lane = jnp.arange(_L, dtype=jnp.int32)

    pltpu.async_copy(magn_hbm.at[t, b], magn_v, sem_m)
    pltpu.async_copy(sls_hbm.at[t, b], sls_v, sem_s)
    pltpu.make_async_copy(magn_hbm.at[t, b], magn_v, sem_m).wait()

    def _argmax_pop():
        def body(i, carry):
            mv, mi = carry
            base = i * (_UNROLL * _L)
            for j in range(_UNROLL):
                v = magn_v[pl.ds(base + j * _L, _L)]
                gi = base + j * _L + lane
                better = v > mv
                mv = jnp.where(better, v, mv)
                mi = jnp.where(better, gi, mi)
            return mv, mi
        mv, mi = lax.fori_loop(
            0, _T // (_UNROLL * _L), body,
            (jnp.full((_L,), _NEG, jnp.float32), jnp.zeros((_L,), jnp.int32)))
        m = jnp.max(mv)
        gidx = jnp.min(jnp.where(mv == m, mi, _BIGI))
        plsc.store_scatter(magn_v, [jnp.full((_L,), gidx, jnp.int32)],
                           jnp.full((_L,), _NEG, jnp.float32),
                           mask=lane == 0)
        return gidx

    i0 = _argmax_pop()
    i1 = _argmax_pop()
    i2 = _argmax_pop()

    r0 = b * _T
    r1 = (_B + b) * _T
    fidx = jnp.where(lane == 0, r0 + i0,
           jnp.where(lane == 1, r0 + i1,
           jnp.where(lane == 2, r0 + i2,
           jnp.where(lane == 3, r1 + i0,
           jnp.where(lane == 4, r1 + i1,
           jnp.where(lane == 5, r1 + i2, r0 + i0))))))
    idx_v[...] = fidx

    cp_a = pltpu.async_copy(afl_hbm.at[idx_v], rows_a, sem_a)
    cp_n = pltpu.async_copy(nfl_hbm.at[idx_v], rows_n, sem_n)
    cp_a.wait()
    cp_n.wait()

    pltpu.make_async_copy(sls_hbm.at[t, b], sls_v, sem_s).wait()
    sls_sel = plsc.load_gather(
        sls_v, [jnp.where(lane == 0, i0, jnp.where(lane == 1, i1, i2))])
    vls = jnp.sum(jnp.where(lane < _K, sls_sel, 0.0)) * jnp.float32(1.0 / _K)

    tmask = jnp.full((_L,), 0, jnp.int32) + t == 0

    def pick(r, s):
        return jnp.where(tmask, rows_a[r, s], rows_n[r, s])

    third = jnp.float32(1.0 / 3.0)
    acc0 = jnp.zeros((_L,), jnp.float32)
    acc1 = jnp.zeros((_L,), jnp.float32)
    for c in range(_F // _L):
        s = pl.ds(c * _L, _L)
        m0 = (pick(0, s) + pick(1, s) + pick(2, s)) * third
        acc0 = acc0 + m0 * m0
        m1 = (pick(3, s) + pick(4, s) + pick(5, s)) * third
        acc1 = acc1 + m1 * m1
    ssq0 = jnp.sum(acc0)
    ssq1 = jnp.sum(acc1)

    out_v[...] = jnp.where(lane == 0, ssq0,
                 jnp.where(lane == 1, ssq1,
                 jnp.where(lane == 2, vls, 0.0)))
    pltpu.sync_copy(out_v, out_hbm.at[t, b])


_sc_call = functools.partial(
    pl.kernel,
    mesh=plsc.VectorSubcoreMesh(core_axis_name="c", subcore_axis_name="s",
                                num_cores=2, num_subcores=16),
    out_type=jax.ShapeDtypeStruct((_NC, _B, _L), jnp.float32),
    scratch_types=[
        pltpu.VMEM((_T,), jnp.float32),
        pltpu.VMEM((_T,), jnp.float32),
        pltpu.VMEM((_L,), jnp.int32),
        pltpu.VMEM((_L, _F), jnp.float32),
        pltpu.VMEM((_L, _F), jnp.float32),
        pltpu.VMEM((_L,), jnp.float32),
        pltpu.SemaphoreType.DMA,
        pltpu.SemaphoreType.DMA,
        pltpu.SemaphoreType.DMA,
        pltpu.SemaphoreType.DMA,
    ],
    compiler_params=pltpu.CompilerParams(needs_layout_passes=False),
)(_sc_body)


def _tc_body(x_ref, o_ref):
    x = x_ref[...]
    la = jnp.abs(_MARGIN - jnp.sqrt(x[0, :, 0:2]))
    ln = jnp.sqrt(x[1, :, 0:2])
    loss_rtfm = jnp.mean((la + ln) ** 2)
    vls_abn = x[0, :, 2]
    vls_norm = x[1, :, 2]
    bcea = -jnp.mean(jnp.maximum(jnp.log(vls_abn), -100.0))
    bcen = -jnp.mean(jnp.maximum(jnp.log(1.0 - vls_norm), -100.0))
    o_ref[0] = _ALPHA * loss_rtfm
    o_ref[1] = bcea + bcen


_tc_call = pl.pallas_call(
    _tc_body,
    out_shape=jax.ShapeDtypeStruct((2,), jnp.float32),
    out_specs=pl.BlockSpec(memory_space=pltpu.SMEM),
)


def kernel(abnr_feat_magn, norm_feat_magn, abnr_feats, norm_feats,
           abnr_sls, norm_sls, ldata):
    magn = jnp.stack([abnr_feat_magn, norm_feat_magn])
    sls = jnp.stack([abnr_sls, norm_sls])
    afl = abnr_feats.reshape(_NC * _B * _T, _F)
    nfl = norm_feats.reshape(_NC * _B * _T, _F)
    part = _sc_call(magn, sls, afl, nfl)
    return _tc_call(part)
